# baseline (speedup 1.0000x reference)
import functools

import jax
import jax.numpy as jnp
from jax import lax
from jax.experimental import pallas as pl
from jax.experimental.pallas import tpu as pltpu
from jax.experimental.pallas import tpu_sc as plsc

IN_SPACE = 1000000
DIM = 16
BATCH = 16384
HIST = 50

TOTAL = BATCH * HIST
G_PAD = 1 << 20
CBLK = 262144

NUM_WORKERS = 32
CHUNK = 128
PER_W = TOTAL // NUM_WORKERS
NCHUNK = PER_W // CHUNK
FIRE_K = 8


def _stage1_body(x_ref, w_ref, b_ref, o_ref):
    x = x_ref[...]
    w = w_ref[...].reshape(1, DIM)
    y = lax.dot_general(w, x, (((1,), (0,)), ((), ())),
                        preferred_element_type=jnp.float32)
    o_ref[...] = jax.nn.sigmoid(y + b_ref[0]).reshape(CBLK)


def _stage1(table_t, w, b):
    return pl.pallas_call(
        _stage1_body,
        grid=(G_PAD // CBLK,),
        in_specs=[
            pl.BlockSpec((DIM, CBLK), lambda i: (0, i)),
            pl.BlockSpec((DIM, 1), lambda i: (0, 0)),
            pl.BlockSpec(memory_space=pltpu.SMEM),
        ],
        out_specs=pl.BlockSpec((CBLK,), lambda i: (i,)),
        out_shape=jax.ShapeDtypeStruct((G_PAD,), jnp.float32),
    )(table_t, w, b)


SLICE = G_PAD // 16


def _gather_body(g_hbm, idx_hbm, out_hbm, idx_v, vals_v, g_spmem, sem):
    cid = lax.axis_index("c")
    sid = lax.axis_index("s")
    wid = sid * 2 + cid
    idx_cp = pltpu.async_copy(idx_hbm.at[wid], idx_v, sem)
    pltpu.sync_copy(g_hbm.at[pl.ds(sid * SLICE, SLICE)],
                    g_spmem.at[pl.ds(sid * SLICE, SLICE)])
    idx_cp.wait()
    plsc.subcore_barrier()

    def fire(base):
        for t in range(FIRE_K):
            pltpu.async_copy(g_spmem.at[idx_v.at[base + t]],
                             vals_v.at[base + t], sem)

    def drain(base):
        for t in range(FIRE_K):
            pltpu.make_async_copy(g_hbm.at[idx_v.at[base + t]],
                                  vals_v.at[base + t], sem).wait()

    fire(0)
    fire(FIRE_K)

    def group(jo, carry):
        fire(jo * FIRE_K)
        drain((jo - 2) * FIRE_K)
        return carry

    lax.fori_loop(2, NCHUNK // FIRE_K, group, 0)
    drain(NCHUNK - 2 * FIRE_K)
    drain(NCHUNK - FIRE_K)
    pltpu.sync_copy(vals_v, out_hbm.at[wid])


_gather = functools.partial(
    pl.kernel,
    out_type=jax.ShapeDtypeStruct((NUM_WORKERS, NCHUNK, CHUNK), jnp.float32),
    mesh=plsc.VectorSubcoreMesh(core_axis_name="c", subcore_axis_name="s"),
    scratch_types=[
        pltpu.VMEM((NCHUNK, CHUNK), jnp.int32),
        pltpu.VMEM((NCHUNK, CHUNK), jnp.float32),
        pltpu.VMEM_SHARED((G_PAD,), jnp.float32),
        pltpu.SemaphoreType.DMA,
    ],
)(_gather_body)


@jax.jit
def kernel(indices, table, W, b):
    g = _stage1(table.T, W, b)
    idx3 = indices.astype(jnp.int32).T.reshape(NUM_WORKERS, NCHUNK, CHUNK)
    out3 = _gather(g, idx3)
    return out3.reshape(HIST, 1, BATCH).transpose(2, 0, 1)


# speedup vs baseline: 1.0126x; 1.0126x over previous
import functools

import jax
import jax.numpy as jnp
from jax import lax
from jax.experimental import pallas as pl
from jax.experimental.pallas import tpu as pltpu
from jax.experimental.pallas import tpu_sc as plsc

IN_SPACE = 1000000
DIM = 16
BATCH = 16384
HIST = 50

TOTAL = BATCH * HIST
G_PAD = 1 << 20
CBLK = 131072

NUM_WORKERS = 32
CHUNK = 128
PER_W = TOTAL // NUM_WORKERS
NCHUNK = PER_W // CHUNK
FIRE_K = 4


def _stage1_body(x_ref, w_ref, b_ref, o_ref):
    x = x_ref[...]
    w = w_ref[...].reshape(1, DIM)
    y = lax.dot_general(w, x, (((1,), (0,)), ((), ())),
                        preferred_element_type=jnp.float32)
    o_ref[...] = jax.nn.sigmoid(y + b_ref[0]).reshape(CBLK)


def _stage1(table_t, w, b):
    return pl.pallas_call(
        _stage1_body,
        grid=(G_PAD // CBLK,),
        in_specs=[
            pl.BlockSpec((DIM, CBLK), lambda i: (0, i)),
            pl.BlockSpec((DIM, 1), lambda i: (0, 0)),
            pl.BlockSpec(memory_space=pltpu.SMEM),
        ],
        out_specs=pl.BlockSpec((CBLK,), lambda i: (i,)),
        out_shape=jax.ShapeDtypeStruct((G_PAD,), jnp.float32),
    )(table_t, w, b)


SLICE = G_PAD // 16


def _gather_body(g_hbm, idx_hbm, out_hbm, idx_v, vals_v, g_spmem, sem):
    cid = lax.axis_index("c")
    sid = lax.axis_index("s")
    wid = sid * 2 + cid
    idx_cp = pltpu.async_copy(idx_hbm.at[wid], idx_v, sem)
    pltpu.sync_copy(g_hbm.at[pl.ds(sid * SLICE, SLICE)],
                    g_spmem.at[pl.ds(sid * SLICE, SLICE)])
    idx_cp.wait()
    plsc.subcore_barrier()

    def fire(base):
        for t in range(FIRE_K):
            pltpu.async_copy(g_spmem.at[idx_v.at[base + t]],
                             vals_v.at[base + t], sem)

    def drain(base):
        for t in range(FIRE_K):
            pltpu.make_async_copy(g_hbm.at[idx_v.at[base + t]],
                                  vals_v.at[base + t], sem).wait()

    fire(0)
    fire(FIRE_K)

    def group(jo, carry):
        fire(jo * FIRE_K)
        drain((jo - 2) * FIRE_K)
        return carry

    lax.fori_loop(2, NCHUNK // FIRE_K, group, 0)
    drain(NCHUNK - 2 * FIRE_K)
    drain(NCHUNK - FIRE_K)
    pltpu.sync_copy(vals_v, out_hbm.at[wid])


_gather = functools.partial(
    pl.kernel,
    out_type=jax.ShapeDtypeStruct((NUM_WORKERS, NCHUNK, CHUNK), jnp.float32),
    mesh=plsc.VectorSubcoreMesh(core_axis_name="c", subcore_axis_name="s"),
    scratch_types=[
        pltpu.VMEM((NCHUNK, CHUNK), jnp.int32),
        pltpu.VMEM((NCHUNK, CHUNK), jnp.float32),
        pltpu.VMEM_SHARED((G_PAD,), jnp.float32),
        pltpu.SemaphoreType.DMA,
    ],
)(_gather_body)


@jax.jit
def kernel(indices, table, W, b):
    g = _stage1(table.T, W, b)
    idx3 = indices.astype(jnp.int32).T.reshape(NUM_WORKERS, NCHUNK, CHUNK)
    out3 = _gather(g, idx3)
    return out3.reshape(HIST, 1, BATCH).transpose(2, 0, 1)


# speedup vs baseline: 1.0499x; 1.0368x over previous
import functools

import jax
import jax.numpy as jnp
from jax import lax
from jax.experimental import pallas as pl
from jax.experimental.pallas import tpu as pltpu
from jax.experimental.pallas import tpu_sc as plsc

IN_SPACE = 1000000
DIM = 16
BATCH = 16384
HIST = 50

TOTAL = BATCH * HIST
G_PAD = 1 << 20
CBLK = 131072

NUM_WORKERS = 32
CHUNK = 128
PER_W = TOTAL // NUM_WORKERS
NCHUNK = PER_W // CHUNK
FIRE_K = 4


IDX_ROWS = TOTAL // CHUNK
IDX_PAD = 8192


def _stage1_body(x_ref, w_ref, b_ref, idxt_ref, o_ref, oidx_ref):
    x = x_ref[...]
    w = w_ref[...].reshape(1, DIM)
    y = lax.dot_general(w, x, (((1,), (0,)), ((), ())),
                        preferred_element_type=jnp.float32)
    o_ref[...] = jax.nn.sigmoid(y + b_ref[0]).reshape(CBLK)

    @pl.when(pl.program_id(0) == 0)
    def _():
        v = idxt_ref[...]
        oidx_ref[pl.ds(0, IDX_ROWS), :] = v.reshape(IDX_ROWS, CHUNK)


def _stage1(table_t, w, b, idx_t):
    return pl.pallas_call(
        _stage1_body,
        grid=(G_PAD // CBLK,),
        in_specs=[
            pl.BlockSpec((DIM, CBLK), lambda i: (0, i)),
            pl.BlockSpec((DIM, 1), lambda i: (0, 0)),
            pl.BlockSpec(memory_space=pltpu.SMEM),
            pl.BlockSpec((HIST, BATCH), lambda i: (0, 0)),
        ],
        out_specs=[
            pl.BlockSpec((CBLK,), lambda i: (i,)),
            pl.BlockSpec((IDX_PAD, CHUNK), lambda i: (0, 0)),
        ],
        out_shape=[
            jax.ShapeDtypeStruct((G_PAD,), jnp.float32),
            jax.ShapeDtypeStruct((IDX_PAD, CHUNK), jnp.int32),
        ],
    )(table_t, w, b, idx_t)


SLICE = G_PAD // 16


def _gather_body(g_hbm, idx_hbm, out_hbm, idx_v, vals_v, g_spmem, sem):
    cid = lax.axis_index("c")
    sid = lax.axis_index("s")
    wid = sid * 2 + cid
    idx_cp = pltpu.async_copy(idx_hbm.at[pl.ds(wid * NCHUNK, NCHUNK)],
                              idx_v, sem)
    pltpu.sync_copy(g_hbm.at[pl.ds(sid * SLICE, SLICE)],
                    g_spmem.at[pl.ds(sid * SLICE, SLICE)])
    idx_cp.wait()
    plsc.subcore_barrier()

    def fire(base):
        for t in range(FIRE_K):
            pltpu.async_copy(g_spmem.at[idx_v.at[base + t]],
                             vals_v.at[base + t], sem)

    def drain(base):
        for t in range(FIRE_K):
            pltpu.make_async_copy(g_hbm.at[idx_v.at[base + t]],
                                  vals_v.at[base + t], sem).wait()

    fire(0)
    fire(FIRE_K)

    def group(jo, carry):
        fire(jo * FIRE_K)
        drain((jo - 2) * FIRE_K)
        return carry

    lax.fori_loop(2, NCHUNK // FIRE_K, group, 0)
    drain(NCHUNK - 2 * FIRE_K)
    drain(NCHUNK - FIRE_K)
    pltpu.sync_copy(vals_v, out_hbm.at[wid])


_gather = functools.partial(
    pl.kernel,
    out_type=jax.ShapeDtypeStruct((NUM_WORKERS, NCHUNK, CHUNK), jnp.float32),
    mesh=plsc.VectorSubcoreMesh(core_axis_name="c", subcore_axis_name="s"),
    scratch_types=[
        pltpu.VMEM((NCHUNK, CHUNK), jnp.int32),
        pltpu.VMEM((NCHUNK, CHUNK), jnp.float32),
        pltpu.VMEM_SHARED((G_PAD,), jnp.float32),
        pltpu.SemaphoreType.DMA,
    ],
)(_gather_body)


@jax.jit
def kernel(indices, table, W, b):
    idx_t = indices.astype(jnp.int32).T
    g, idx2 = _stage1(table.T, W, b, idx_t)
    out3 = _gather(g, idx2)
    return out3.reshape(HIST, 1, BATCH).transpose(2, 0, 1)


# speedup vs baseline: 1.0730x; 1.0220x over previous
import functools

import jax
import jax.numpy as jnp
from jax import lax
from jax.experimental import pallas as pl
from jax.experimental.pallas import tpu as pltpu
from jax.experimental.pallas import tpu_sc as plsc

IN_SPACE = 1000000
DIM = 16
BATCH = 16384
HIST = 50

TOTAL = BATCH * HIST
G_PAD = 1 << 20
CBLK = 131072

NUM_WORKERS = 32
CHUNK = 128
PER_W = TOTAL // NUM_WORKERS
NCHUNK = PER_W // CHUNK
FIRE_K = 4


IDX_ROWS = TOTAL // CHUNK
IDX_PAD = 8192


def _stage1_body(x_ref, w_ref, b_ref, idxt_ref, o_ref, oidx_ref):
    x = x_ref[...]
    w = jnp.stack([w_ref[0, j] for j in range(DIM)]).reshape(1, DIM)
    y = lax.dot_general(w, x, (((1,), (0,)), ((), ())),
                        preferred_element_type=jnp.float32)
    o_ref[...] = jax.nn.sigmoid(y + b_ref[0]).reshape(CBLK)

    @pl.when(pl.program_id(0) == 0)
    def _():
        v = idxt_ref[...]
        oidx_ref[pl.ds(0, IDX_ROWS), :] = v.reshape(IDX_ROWS, CHUNK)


def _stage1(table_t, w, b, idx_t):
    return pl.pallas_call(
        _stage1_body,
        grid=(G_PAD // CBLK,),
        in_specs=[
            pl.BlockSpec((DIM, CBLK), lambda i: (0, i)),
            pl.BlockSpec(memory_space=pltpu.SMEM),
            pl.BlockSpec(memory_space=pltpu.SMEM),
            pl.BlockSpec((HIST, BATCH), lambda i: (0, 0)),
        ],
        out_specs=[
            pl.BlockSpec((CBLK,), lambda i: (i,)),
            pl.BlockSpec((IDX_PAD, CHUNK), lambda i: (0, 0)),
        ],
        out_shape=[
            jax.ShapeDtypeStruct((G_PAD,), jnp.float32),
            jax.ShapeDtypeStruct((IDX_PAD, CHUNK), jnp.int32),
        ],
    )(table_t, w, b, idx_t)


SLICE = G_PAD // 16


def _gather_body(g_hbm, idx_hbm, out_hbm, idx_v, vals_v, g_spmem, sem):
    cid = lax.axis_index("c")
    sid = lax.axis_index("s")
    wid = sid * 2 + cid
    idx_cp = pltpu.async_copy(idx_hbm.at[pl.ds(wid * NCHUNK, NCHUNK)],
                              idx_v, sem)
    pltpu.sync_copy(g_hbm.at[pl.ds(sid * SLICE, SLICE)],
                    g_spmem.at[pl.ds(sid * SLICE, SLICE)])
    idx_cp.wait()
    plsc.subcore_barrier()

    def fire(base):
        for t in range(FIRE_K):
            pltpu.async_copy(g_spmem.at[idx_v.at[base + t]],
                             vals_v.at[base + t], sem)

    def drain(base):
        for t in range(FIRE_K):
            pltpu.make_async_copy(g_hbm.at[idx_v.at[base + t]],
                                  vals_v.at[base + t], sem).wait()

    fire(0)
    fire(FIRE_K)

    def group(jo, carry):
        fire(jo * FIRE_K)
        drain((jo - 2) * FIRE_K)
        return carry

    lax.fori_loop(2, NCHUNK // FIRE_K, group, 0)
    drain(NCHUNK - 2 * FIRE_K)
    drain(NCHUNK - FIRE_K)
    pltpu.sync_copy(vals_v, out_hbm.at[wid])


_gather = functools.partial(
    pl.kernel,
    out_type=jax.ShapeDtypeStruct((NUM_WORKERS, NCHUNK, CHUNK), jnp.float32),
    mesh=plsc.VectorSubcoreMesh(core_axis_name="c", subcore_axis_name="s"),
    scratch_types=[
        pltpu.VMEM((NCHUNK, CHUNK), jnp.int32),
        pltpu.VMEM((NCHUNK, CHUNK), jnp.float32),
        pltpu.VMEM_SHARED((G_PAD,), jnp.float32),
        pltpu.SemaphoreType.DMA,
    ],
)(_gather_body)


@jax.jit
def kernel(indices, table, W, b):
    idx_t = indices.astype(jnp.int32).T
    g, idx2 = _stage1(table.T, W.T, b, idx_t)
    out3 = _gather(g, idx2)
    return out3.reshape(HIST, 1, BATCH).transpose(2, 0, 1)
